# Initial kernel scaffold; baseline (speedup 1.0000x reference)
import jax
import jax.numpy as jnp
from jax.experimental import pallas as pl


def kernel(x, indices, weight):
    raise NotImplementedError("write your pallas kernel here")


# speedup vs baseline: 4.8988x; 4.8988x over previous
import functools

import jax
import jax.numpy as jnp
from jax import lax
from jax.experimental import pallas as pl
from jax.experimental.pallas import tpu as pltpu
from jax.experimental.pallas import tpu_sc as plsc

NUM_EMBEDDINGS = 100000
EMBED = 64
BATCH = 4096
FIELDS = 100

NC = 2
NS = 16
NW = NC * NS

TOTAL = BATCH * FIELDS
CHUNK = 128
ROWS_PER_W = TOTAL // NW
NCH = ROWS_PER_W // CHUNK
NBUF = 4


def _gather_body(idx_hbm, w_hbm, out_hbm, idx_v, bufs, gsems, psems):
    c = lax.axis_index("c")
    s = lax.axis_index("s")
    wid = s * NC + c

    pltpu.sync_copy(idx_hbm.at[wid], idx_v)

    def start_gather(j, b):
        pltpu.async_copy(w_hbm.at[idx_v.at[j]], bufs.at[b], gsems.at[b])

    def wait_gather(b):
        pltpu.make_async_copy(w_hbm.at[idx_v.at[0]], bufs.at[b], gsems.at[b]).wait()

    def start_put(j, b):
        pltpu.async_copy(bufs.at[b], out_hbm.at[wid, j], psems.at[b])

    def wait_put(b):
        pltpu.make_async_copy(bufs.at[b], out_hbm.at[wid, 0], psems.at[b]).wait()

    for b in range(NBUF):
        start_gather(b, b)

    @pl.loop(0, NCH - NBUF, step=NBUF)
    def _steady(g):
        for b in range(NBUF):
            j = g + b
            wait_gather(b)
            start_put(j, b)
            wait_put(b)
            start_gather(j + NBUF, b)

    for b in range(NBUF):
        wait_gather(b)
        start_put(NCH - NBUF + b, b)
    for b in range(NBUF):
        wait_put(b)


@jax.jit
def _gather_sc(idx, weight):
    mesh = plsc.VectorSubcoreMesh(core_axis_name="c", subcore_axis_name="s")
    fn = pl.kernel(
        _gather_body,
        out_type=jax.ShapeDtypeStruct((NW, NCH, CHUNK, EMBED), jnp.float32),
        mesh=mesh,
        scratch_types=[
            pltpu.VMEM((NCH, CHUNK), jnp.int32),
            pltpu.VMEM((NBUF, CHUNK, EMBED), jnp.float32),
            pltpu.SemaphoreType.DMA((NBUF,)),
            pltpu.SemaphoreType.DMA((NBUF,)),
        ],
        compiler_params=pltpu.CompilerParams(use_tc_tiling_on_sc=False),
    )
    return fn(idx, weight)


def kernel(x, indices, weight):
    idx = indices.astype(jnp.int32).reshape(NW, NCH, CHUNK)
    out = _gather_sc(idx, weight)
    return out.reshape(BATCH, FIELDS, EMBED)
